# Initial kernel scaffold; baseline (speedup 1.0000x reference)
import functools

import jax
import jax.numpy as jnp
from jax import lax
from jax.experimental import pallas as pl
from jax.experimental.pallas import tpu as pltpu
from jax.experimental.pallas import tpu_sc as plsc

T = 8192
E = 8
D = 1024
C = 1024
L = 16
NC = 2
NS = 16
TPT = T // NS
GROUPS = TPT // L
E_PER_CORE = E // NC
SEL_CAP = TPT + L
CH = L
DBLK = D // L
ROWS_PT = C // NS

_mesh = plsc.VectorSubcoreMesh(core_axis_name="c", subcore_axis_name="s")


@functools.partial(
    pl.kernel,
    out_type=jax.ShapeDtypeStruct((E, C, D), jnp.float32),
    mesh=_mesh,
    scratch_types=[
        pltpu.VMEM((TPT, E), jnp.float32),
        pltpu.VMEM((TPT, E), jnp.int32),
        pltpu.VMEM((TPT,), jnp.float32),
        pltpu.VMEM((SEL_CAP,), jnp.int32),
        pltpu.VMEM((SEL_CAP,), jnp.int32),
        pltpu.VMEM((SEL_CAP,), jnp.float32),
        pltpu.VMEM((CH, D), jnp.float32),
        pltpu.VMEM((CH, D), jnp.float32),
        pltpu.VMEM_SHARED((C, D), jnp.float32),
        pltpu.SemaphoreType.DMA,
    ],
)
def _dispatch(in_hbm, route_hbm, score_hbm, out_hbm,
              score_v, route_v, thr_v, selid_v, selslot_v, selgate_v,
              rows_v, zero_v, acc_sh, sem):
    c = lax.axis_index("c")
    s = lax.axis_index("s")
    tok0 = s * TPT

    pltpu.sync_copy(score_hbm.at[pl.ds(tok0, TPT)], score_v)
    pltpu.sync_copy(route_hbm.at[pl.ds(tok0, TPT)], route_v)

    lanes = lax.iota(jnp.int32, L)
    neg = jnp.full((L,), -jnp.inf, jnp.float32)

    def _zrow(r, carry):
        for k in range(DBLK):
            zero_v[r, pl.ds(k * L, L)] = jnp.zeros((L,), jnp.float32)
        return carry
    lax.fori_loop(0, CH, _zrow, 0)

    def _thr(g, carry):
        tl = g * L + lanes
        sc = [plsc.load_gather(score_v, [tl, jnp.full((L,), e, jnp.int32)])
              for e in range(E)]
        m1 = sc[0]
        for e in range(1, E):
            m1 = jnp.maximum(m1, sc[e])
        excl = jnp.zeros((L,), jnp.bool_)
        m2 = neg
        for e in range(E):
            is_first = (sc[e] == m1) & (~excl)
            m2 = jnp.maximum(m2, jnp.where(is_first, neg, sc[e]))
            excl = excl | is_first
        thr_v[pl.ds(g * L, L)] = m2
        return carry
    lax.fori_loop(0, GROUPS, _thr, 0)

    def _phase(p, carry):
        expert = c * E_PER_CORE + p

        for j in range(ROWS_PT // CH):
            pltpu.sync_copy(zero_v, acc_sh.at[pl.ds(s * ROWS_PT + j * CH, CH)])
        plsc.subcore_barrier()

        def _pre(i, cc):
            selid_v[pl.ds(i * L, L)] = jnp.zeros((L,), jnp.int32)
            selslot_v[pl.ds(i * L, L)] = jnp.zeros((L,), jnp.int32)
            selgate_v[pl.ds(i * L, L)] = jnp.zeros((L,), jnp.float32)
            return cc
        lax.fori_loop(0, SEL_CAP // L, _pre, 0)

        ecol = jnp.full((L,), expert, jnp.int32)

        def _compact(g, cnt):
            tl = g * L + lanes
            sce = plsc.load_gather(score_v, [tl, ecol])
            th = thr_v[pl.ds(g * L, L)]
            gate = jnp.where(sce >= th, sce, jnp.zeros((L,), jnp.float32))
            msk = gate > 0.0
            slot = plsc.load_gather(route_v, [tl, ecol])
            plsc.store_compressed(selid_v.at[pl.ds(cnt, L)], tok0 + tl, mask=msk)
            plsc.store_compressed(selslot_v.at[pl.ds(cnt, L)], slot, mask=msk)
            plsc.store_compressed(selgate_v.at[pl.ds(cnt, L)], gate, mask=msk)
            return cnt + jnp.sum(msk.astype(jnp.int32))
        n = lax.fori_loop(0, GROUPS, _compact, jnp.int32(0))

        def _chunk(j, cc):
            off = j * CH
            tid = selid_v[pl.ds(off, CH)]
            slot = selslot_v[pl.ds(off, CH)]
            pltpu.async_copy(in_hbm.at[tid], rows_v, sem).wait()

            def _scale(r, c2):
                g = plsc.load_gather(
                    selgate_v, [jnp.full((L,), off + r, jnp.int32)])
                for k in range(DBLK):
                    rows_v[r, pl.ds(k * L, L)] = rows_v[r, pl.ds(k * L, L)] * g
                return c2
            lax.fori_loop(0, CH, _scale, 0)

            pltpu.sync_copy(rows_v, acc_sh.at[slot], add=True)
            return cc
        lax.fori_loop(0, (n + CH - 1) // CH, _chunk, 0)

        plsc.subcore_barrier()

        pltpu.sync_copy(acc_sh.at[pl.ds(s * ROWS_PT, ROWS_PT)],
                        out_hbm.at[expert, pl.ds(s * ROWS_PT, ROWS_PT)])
        return carry
    lax.fori_loop(0, E_PER_CORE, _phase, 0)


def kernel(in_flow, route_indices, loads, capacities, score):
    del loads, capacities
    return _dispatch(in_flow, route_indices, score)


# speedup vs baseline: 1.7077x; 1.7077x over previous
import functools

import jax
import jax.numpy as jnp
from jax import lax
from jax.experimental import pallas as pl
from jax.experimental.pallas import tpu as pltpu
from jax.experimental.pallas import tpu_sc as plsc

T = 8192
E = 8
D = 1024
C = 1024
L = 16
NC = 2
NS = 16
TPT = T // NS
GROUPS = T // L
E_PER_CORE = E // NC
OWN = C // NS
CH = L
DBLK = D // L
NRING = 1024
NRROW = NRING // CH
DRAIN_AT = 544

_mesh = plsc.VectorSubcoreMesh(core_axis_name="c", subcore_axis_name="s")

_DNUMS = lax.GatherDimensionNumbers(
    offset_dims=(), collapsed_slice_dims=(0,), start_index_map=(0,))


def _splat(vec, r):
    idx = jnp.full((L, 1), r, jnp.int32)
    return lax.gather(vec, idx, _DNUMS, (1,),
                      mode=lax.GatherScatterMode.PROMISE_IN_BOUNDS)


@functools.partial(
    pl.kernel,
    out_type=jax.ShapeDtypeStruct((E, C, D), jnp.float32),
    mesh=_mesh,
    scratch_types=[
        pltpu.VMEM((E, TPT), jnp.float32),
        pltpu.VMEM((T,), jnp.float32),
        pltpu.VMEM((T,), jnp.float32),
        pltpu.VMEM((T,), jnp.int32),
        pltpu.VMEM((NRROW + 1, CH), jnp.int32),
        pltpu.VMEM((NRROW + 1, CH), jnp.int32),
        pltpu.VMEM((NRING + L,), jnp.float32),
        pltpu.VMEM((CH, D), jnp.float32),
        pltpu.VMEM((OWN, D), jnp.float32),
        pltpu.VMEM_SHARED((T,), jnp.float32),
        pltpu.SemaphoreType.DMA,
    ],
    compiler_params=pltpu.CompilerParams(needs_layout_passes=False),
)
def _dispatch(in_hbm, route_hbm, score_hbm, out_hbm,
              score_sl, thr_v, score_col, route_col,
              selid_v, selslot_v, selgate_v, rows_v, acc_v, thr_sh, sem):
    c = lax.axis_index("c")
    s = lax.axis_index("s")
    tok0 = s * TPT

    lanes = lax.iota(jnp.int32, L)
    neg = jnp.full((L,), -jnp.inf, jnp.float32)
    zf = jnp.zeros((L,), jnp.float32)
    zi = jnp.zeros((L,), jnp.int32)

    def _pre(i, cc):
        selid_v[i, :] = zi
        selslot_v[i, :] = zi
        return cc
    lax.fori_loop(0, NRROW + 1, _pre, 0)

    pltpu.sync_copy(score_hbm.at[:, pl.ds(tok0, TPT)], score_sl)

    def _thr(g, cc):
        sc = [score_sl[e, pl.ds(g * L, L)] for e in range(E)]
        m1 = sc[0]
        for e in range(1, E):
            m1 = jnp.maximum(m1, sc[e])
        excl = jnp.zeros((L,), jnp.bool_)
        m2 = neg
        for e in range(E):
            is_first = (sc[e] == m1) & (~excl)
            m2 = jnp.maximum(m2, jnp.where(is_first, neg, sc[e]))
            excl = excl | is_first
        thr_v[pl.ds(tok0 + g * L, L)] = m2
        return cc
    lax.fori_loop(0, TPT // L, _thr, 0)

    pltpu.sync_copy(thr_v.at[pl.ds(tok0, TPT)], thr_sh.at[pl.ds(tok0, TPT)])
    plsc.subcore_barrier()
    pltpu.sync_copy(thr_sh, thr_v)

    slot_base = s * OWN

    def _drain_one(done):
        jr = lax.shift_right_logical(done, 4) & (NRROW - 1)
        pltpu.async_copy(in_hbm.at[selid_v.at[jr]], rows_v, sem).wait()
        gatev = selgate_v[pl.ds(done & (NRING - 1), CH)]
        slotv = selslot_v[jr, :]

        def _acc(r, cc):
            g = _splat(gatev, r)
            sl = _splat(slotv, r)
            for k in range(DBLK):
                contrib = rows_v[r, pl.ds(k * L, L)] * g
                plsc.addupdate_scatter(acc_v, [sl, k * L + lanes], contrib)
            return cc
        lax.fori_loop(0, CH, _acc, 0)
        return done + CH

    def _phase(expert):
        pltpu.sync_copy(score_hbm.at[expert], score_col)
        pltpu.sync_copy(route_hbm.at[expert], route_col)

        def _zero(r, cc):
            for k in range(DBLK):
                acc_v[r, pl.ds(k * L, L)] = zf
            return cc
        lax.fori_loop(0, OWN, _zero, 0)

        def _scan(g, carry):
            cnt, done = carry
            sce = score_col[pl.ds(g * L, L)]
            th = thr_v[pl.ds(g * L, L)]
            gate = jnp.where(sce >= th, sce, zf)
            slot = route_col[pl.ds(g * L, L)] - slot_base
            msk = ((gate > 0.0) & (slot >= 0)
                   & (slot < jnp.full((L,), OWN, jnp.int32)))
            inc = msk.astype(jnp.int32)
            raw = cnt + plsc.cumsum(inc) - 1
            pos = jnp.where(msk, raw & (NRING - 1), NRING + lanes)
            phi = lax.shift_right_logical(pos, 4)
            plo = pos & (CH - 1)
            plsc.store_scatter(selid_v, [phi, plo], g * L + lanes)
            plsc.store_scatter(selslot_v, [phi, plo], slot)
            plsc.store_scatter(selgate_v, [pos], gate)
            cnt = cnt + jnp.sum(inc)
            done = lax.cond(cnt - done >= DRAIN_AT, _drain_one,
                            lambda d: d, done)
            return cnt, done
        cnt, done = lax.fori_loop(0, GROUPS, _scan,
                                  (jnp.int32(0), jnp.int32(0)))

        zpos = (cnt + lanes) & (NRING - 1)
        plsc.store_scatter(selgate_v, [zpos], zf)
        nleft = lax.shift_right_logical(cnt - done + CH - 1, 4)

        def _fin(i, done):
            return _drain_one(done)
        lax.fori_loop(0, nleft, _fin, done)

        pltpu.sync_copy(acc_v, out_hbm.at[expert, pl.ds(slot_base, OWN)])

    @pl.when(c == 0)
    def _():
        for expert in range(E_PER_CORE):
            _phase(expert)

    @pl.when(c == 1)
    def _():
        for expert in range(E_PER_CORE, E):
            _phase(expert)


def kernel(in_flow, route_indices, loads, capacities, score):
    del loads, capacities
    return _dispatch(in_flow, route_indices.T, score.T)
